# baseline (speedup 1.0000x reference)
import jax
import jax.numpy as jnp
from jax import lax
from jax.experimental import pallas as pl
from jax.experimental.pallas import tpu as pltpu

_N = 4 * 8192
_D = 32
_H = 16
_Q = 4
_R = _N // _Q
_BLK = 2048


def _tc_body(x_ref, w1s_ref, b1s_ref, w2s_ref, b2_ref, o_ref):
    x = x_ref[...]
    ht = lax.dot_general(w1s_ref[...], x, (((1,), (1,)), ((), ())),
                         preferred_element_type=jnp.float32)
    ht = jnp.maximum(ht + b1s_ref[...], 0.0)
    zt = lax.dot_general(w2s_ref[...], ht, (((1,), (0,)), ((), ())),
                         preferred_element_type=jnp.float32)
    z = zt + b2_ref[0]
    o_ref[...] = 1.0 / (1.0 + jnp.exp(-z))


@jax.jit
def _run_tc(x128, w1s, b1s, w2s, b2):
    return pl.pallas_call(
        _tc_body,
        out_shape=jax.ShapeDtypeStruct((_Q, _R), jnp.float32),
        grid=(_R // _BLK,),
        in_specs=[
            pl.BlockSpec((_BLK, _Q * _D), lambda i: (i, 0)),
            pl.BlockSpec((_Q * _H, _Q * _D), lambda i: (0, 0)),
            pl.BlockSpec((_Q * _H, 1), lambda i: (0, 0)),
            pl.BlockSpec((_Q, _Q * _H), lambda i: (0, 0)),
            pl.BlockSpec(memory_space=pltpu.SMEM),
        ],
        out_specs=pl.BlockSpec((_Q, _BLK), lambda i: (0, i)),
    )(x128, w1s, b1s, w2s, b2)


def kernel(embeddings, W1, b1, W2, b2):
    bsz, seq, _ = embeddings.shape
    x128 = embeddings.transpose(1, 0, 2).reshape(_R, _Q * _D)
    eye = jnp.eye(_Q, dtype=jnp.float32)
    w1s = jnp.kron(eye, W1)
    b1s = jnp.tile(b1, _Q).reshape(_Q * _H, 1)
    w2s = jnp.kron(eye, W2.reshape(1, _H))
    return _run_tc(x128, w1s, b1s, w2s, b2)


# speedup vs baseline: 1.0419x; 1.0419x over previous
import jax
import jax.numpy as jnp
from jax import lax
from jax.experimental import pallas as pl
from jax.experimental.pallas import tpu as pltpu

_N = 4 * 8192
_D = 32
_H = 16
_Q = 4
_R = _N // _Q
_BLK = 8192


def _tc_body(x_ref, w1s_ref, b1s_ref, w2s_ref, b2_ref, o_ref):
    x = x_ref[...]
    ht = lax.dot_general(w1s_ref[...], x, (((1,), (1,)), ((), ())),
                         preferred_element_type=jnp.float32)
    ht = jnp.maximum(ht + b1s_ref[...], 0.0)
    zt = lax.dot_general(w2s_ref[...], ht, (((1,), (0,)), ((), ())),
                         preferred_element_type=jnp.float32)
    z = zt + b2_ref[0]
    o_ref[...] = 1.0 / (1.0 + jnp.exp(-z))


@jax.jit
def _run_tc(x128, w1s, b1s, w2s, b2):
    return pl.pallas_call(
        _tc_body,
        out_shape=jax.ShapeDtypeStruct((_Q, _R), jnp.float32),
        grid=(_R // _BLK,),
        in_specs=[
            pl.BlockSpec((_BLK, _Q * _D), lambda i: (i, 0)),
            pl.BlockSpec((_Q * _H, _Q * _D), lambda i: (0, 0)),
            pl.BlockSpec((_Q * _H, 1), lambda i: (0, 0)),
            pl.BlockSpec((_Q, _Q * _H), lambda i: (0, 0)),
            pl.BlockSpec(memory_space=pltpu.SMEM),
        ],
        out_specs=pl.BlockSpec((_Q, _BLK), lambda i: (0, i)),
    )(x128, w1s, b1s, w2s, b2)


def kernel(embeddings, W1, b1, W2, b2):
    bsz, seq, _ = embeddings.shape
    x128 = embeddings.transpose(1, 0, 2).reshape(_R, _Q * _D)
    eye = jnp.eye(_Q, dtype=jnp.float32)
    w1s = jnp.kron(eye, W1)
    b1s = jnp.tile(b1, _Q).reshape(_Q * _H, 1)
    w2s = jnp.kron(eye, W2.reshape(1, _H))
    return _run_tc(x128, w1s, b1s, w2s, b2)


# speedup vs baseline: 1.3345x; 1.2809x over previous
import jax
import jax.numpy as jnp
from jax import lax
from jax.experimental import pallas as pl
from jax.experimental.pallas import tpu as pltpu

_N = 4 * 8192
_D = 32
_H = 16
_Q = 4
_R = _N // _Q


def _tc_body(x_ref, w1_ref, b1_ref, w2_ref, b2_ref, o_ref, w1s_v, w2s_v):
    w1s_v[...] = jnp.zeros((_Q * _H, _Q * _D), jnp.float32)
    w2s_v[...] = jnp.zeros((_Q, _Q * _H), jnp.float32)
    w1 = w1_ref[...]
    w2 = w2_ref[...]
    for q in range(_Q):
        w1s_v[_H * q:_H * (q + 1), _D * q:_D * (q + 1)] = w1
        w2s_v[q:q + 1, _H * q:_H * (q + 1)] = w2
    b1s = jnp.tile(b1_ref[...], (_Q, 1))

    x = x_ref[...]
    ht = lax.dot_general(w1s_v[...], x, (((1,), (1,)), ((), ())),
                         preferred_element_type=jnp.float32)
    ht = jnp.maximum(ht + b1s, 0.0)
    zt = lax.dot_general(w2s_v[...], ht, (((1,), (0,)), ((), ())),
                         preferred_element_type=jnp.float32)
    z = zt + b2_ref[0]
    o_ref[...] = 1.0 / (1.0 + jnp.exp(-z))


@jax.jit
def _run_tc(x128, w1, b1, w2, b2):
    return pl.pallas_call(
        _tc_body,
        out_shape=jax.ShapeDtypeStruct((_Q, _R), jnp.float32),
        in_specs=[
            pl.BlockSpec((_R, _Q * _D), lambda: (0, 0)),
            pl.BlockSpec((_H, _D), lambda: (0, 0)),
            pl.BlockSpec((_H, 1), lambda: (0, 0)),
            pl.BlockSpec((1, _H), lambda: (0, 0)),
            pl.BlockSpec(memory_space=pltpu.SMEM),
        ],
        out_specs=pl.BlockSpec((_Q, _R), lambda: (0, 0)),
        scratch_shapes=[
            pltpu.VMEM((_Q * _H, _Q * _D), jnp.float32),
            pltpu.VMEM((_Q, _Q * _H), jnp.float32),
        ],
    )(x128, w1, b1, w2, b2)


def kernel(embeddings, W1, b1, W2, b2):
    bsz, seq, _ = embeddings.shape
    x128 = embeddings.transpose(1, 0, 2).reshape(_R, _Q * _D)
    return _run_tc(x128, W1, b1.reshape(_H, 1), W2.reshape(1, _H), b2)


# speedup vs baseline: 1.6055x; 1.2030x over previous
import jax
import jax.numpy as jnp
from jax import lax
from jax.experimental import pallas as pl
from jax.experimental.pallas import tpu as pltpu

_N = 4 * 8192
_D = 32
_H = 16
_Q = 4
_R = _N // _Q


def _tc_body(x_ref, w1_ref, b1_ref, w2_ref, b2_ref, o_ref, w1s_v, w2s_v):
    w1s_v[...] = jnp.zeros((_Q * _H, _Q * _D), jnp.bfloat16)
    w2s_v[...] = jnp.zeros((_Q, _Q * _H), jnp.float32)
    w1 = w1_ref[...].astype(jnp.bfloat16)
    w2 = w2_ref[...]
    for q in range(_Q):
        w1s_v[_H * q:_H * (q + 1), _D * q:_D * (q + 1)] = w1
        w2s_v[q:q + 1, _H * q:_H * (q + 1)] = w2
    b1s = jnp.tile(b1_ref[...], (_Q, 1))

    x = x_ref[...]
    ht = lax.dot_general(w1s_v[...], x, (((1,), (1,)), ((), ())),
                         preferred_element_type=jnp.float32)
    ht = jnp.maximum(ht + b1s, 0.0)
    zt = lax.dot_general(w2s_v[...], ht, (((1,), (0,)), ((), ())),
                         preferred_element_type=jnp.float32)
    z = zt + b2_ref[0]
    o_ref[...] = 1.0 / (1.0 + jnp.exp(-z))


@jax.jit
def _run_tc(x128, w1, b1, w2, b2):
    return pl.pallas_call(
        _tc_body,
        out_shape=jax.ShapeDtypeStruct((_Q, _R), jnp.float32),
        in_specs=[
            pl.BlockSpec((_R, _Q * _D), lambda: (0, 0)),
            pl.BlockSpec((_H, _D), lambda: (0, 0)),
            pl.BlockSpec((_H, 1), lambda: (0, 0)),
            pl.BlockSpec((1, _H), lambda: (0, 0)),
            pl.BlockSpec(memory_space=pltpu.SMEM),
        ],
        out_specs=pl.BlockSpec((_Q, _R), lambda: (0, 0)),
        scratch_shapes=[
            pltpu.VMEM((_Q * _H, _Q * _D), jnp.bfloat16),
            pltpu.VMEM((_Q, _Q * _H), jnp.float32),
        ],
    )(x128, w1, b1, w2, b2)


def kernel(embeddings, W1, b1, W2, b2):
    bsz, seq, _ = embeddings.shape
    x128 = embeddings.transpose(1, 0, 2).reshape(_R, _Q * _D)
    x128 = x128.astype(jnp.bfloat16)
    return _run_tc(x128, W1, b1.reshape(_H, 1), W2.reshape(1, _H), b2)
